# baseline (speedup 1.0000x reference)
import functools

import jax
import jax.numpy as jnp
from jax import lax
from jax.experimental import pallas as pl
from jax.experimental.pallas import tpu as pltpu
from jax.experimental.pallas import tpu_sc as plsc

H = 512
W = 512
PLANE = H * W
ROWS_BUF = 66
BUF = ROWS_BUF * W
PAD = 16
BUFA = BUF + 2 * PAD
NLEV = 3
NBATCH = 4
NSTRIP = 8
NW = 32


def _sc_partials_body(pre_hbm, part_hbm, buf0, buf1, obuf, sem0, sem1):
    cid = lax.axis_index("c")
    sid = lax.axis_index("s")
    wid = sid * 2 + cid
    n = wid // NSTRIP
    strip = wid % NSTRIP
    load_r0 = jnp.minimum(strip * 64, H - ROWS_BUF)
    br0 = jnp.where(strip == NSTRIP - 1, 3, 1)

    bufs = (buf0, buf1)
    sems = (sem0, sem1)

    def start(level):
        plane = ((level * NBATCH + n) * 3 + 2) * PLANE
        base = pl.multiple_of(plane + load_r0 * W, 512)
        return pltpu.async_copy(
            pre_hbm.at[pl.ds(base, BUF)],
            bufs[level % 2].at[pl.ds(PAD, BUF)],
            sems[level % 2],
        )

    iota = lax.iota(jnp.int32, 16)
    lanef = iota.astype(jnp.float32)

    cp = start(0)
    zero = jnp.zeros((16,), jnp.float32)
    SIv = zero
    SJv = zero
    CNTv = zero
    for level in range(NLEV):
        cp.wait()
        if level + 1 < NLEV:
            cp_next = start(level + 1)
        buf = bufs[level % 2]

        def row_body(br, carry, buf=buf):
            a_cnt, a_i, a_jb = carry
            rowf = (load_r0 + br).astype(jnp.float32)
            rb = PAD + br * W
            for k in range(32):
                base = rb + k * 16
                c = buf[pl.ds(base, 16)]
                up = buf[pl.ds(base - W, 16)]
                dn = buf[pl.ds(base + W, 16)]
                lf = buf[pl.ds(base - 1, 16)]
                rt = buf[pl.ds(base + 1, 16)]
                ul = buf[pl.ds(base - W - 1, 16)]
                ur = buf[pl.ds(base - W + 1, 16)]
                dl = buf[pl.ds(base + W - 1, 16)]
                dr = buf[pl.ds(base + W + 1, 16)]
                mx = jnp.maximum(
                    jnp.maximum(jnp.maximum(up, dn), jnp.maximum(lf, rt)),
                    jnp.maximum(jnp.maximum(ul, ur), jnp.maximum(dl, dr)),
                )
                m = c > mx
                if k == 0:
                    m = m & (iota > 0)
                elif k == 31:
                    m = m & (iota < 15)
                mf = jnp.where(m, jnp.float32(1.0), jnp.float32(0.0))
                a_cnt = a_cnt + mf
                a_i = a_i + mf * rowf
                if k > 0:
                    a_jb = a_jb + mf * jnp.float32(16.0 * k)
            return (a_cnt, a_i, a_jb)

        a_cnt, a_i, a_jb = lax.fori_loop(br0, 65, row_body, (zero, zero, zero))
        R = jnp.float32(4.0 * (2 ** level))
        SIv = SIv + R * a_i
        SJv = SJv + R * (a_jb + lanef * a_cnt)
        CNTv = CNTv + a_cnt
        if level + 1 < NLEV:
            cp = cp_next

    obuf[pl.ds(0, 16)] = SIv
    obuf[pl.ds(16, 16)] = SJv
    obuf[pl.ds(32, 16)] = CNTv
    pltpu.sync_copy(obuf, part_hbm.at[wid])


def _make_sc_partials():
    mesh = plsc.VectorSubcoreMesh(
        core_axis_name="c", subcore_axis_name="s", num_cores=2, num_subcores=16
    )
    return pl.kernel(
        _sc_partials_body,
        out_type=jax.ShapeDtypeStruct((NW, 48), jnp.float32),
        mesh=mesh,
        scratch_types=[
            pltpu.VMEM((BUFA,), jnp.float32),
            pltpu.VMEM((BUFA,), jnp.float32),
            pltpu.VMEM((48,), jnp.float32),
            pltpu.SemaphoreType.DMA,
            pltpu.SemaphoreType.DMA,
        ],
    )


def _tc_combine_body(part_ref, t_ref, out_ref):
    p = part_ref[...]
    t = t_ref[...]
    seg = lax.broadcasted_iota(jnp.int32, (NW, 48), 1) // 16
    grp = lax.broadcasted_iota(jnp.int32, (NW, 48), 0) // NSTRIP
    nrow = lax.broadcasted_iota(jnp.int32, (NBATCH, 200), 0)
    cx = (t[:, :, 0] + t[:, :, 2]) * 0.5
    cy = (t[:, :, 1] + t[:, :, 3]) * 0.5

    zero = jnp.float32(0.0)
    off_x = zero
    off_y = zero
    cs_tx = zero
    cs_ty = zero
    ts_tx = zero
    ts_ty = zero
    point_sum = zero
    for nn in range(NBATCH):
        mrow = grp == nn
        si_n = jnp.sum(jnp.where(mrow & (seg == 0), p, 0.0))
        sj_n = jnp.sum(jnp.where(mrow & (seg == 1), p, 0.0))
        c_n = jnp.sum(jnp.where(mrow & (seg == 2), p, 0.0))
        tx_n = jnp.sum(jnp.where(nrow == nn, cx, 0.0))
        ty_n = jnp.sum(jnp.where(nrow == nn, cy, 0.0))
        dx = jnp.abs(si_n - tx_n)
        dy = jnp.abs(sj_n - ty_n)
        off_x = off_x + jnp.where(dx < 1.0, 0.5 * dx * dx, dx - 0.5)
        off_y = off_y + jnp.where(dy < 1.0, 0.5 * dy * dy, dy - 0.5)
        cs_tx = cs_tx + si_n
        cs_ty = cs_ty + sj_n
        ts_tx = ts_tx + tx_n
        ts_ty = ts_ty + ty_n
        point_sum = point_sum + c_n
    loss = (off_x / jnp.abs(off_x) * (cs_tx - ts_tx)
            + off_y / jnp.abs(off_y) * (cs_ty - ts_ty)) / point_sum
    out_ref[0, 0] = loss


def _tc_combine(part, target):
    return pl.pallas_call(
        _tc_combine_body,
        out_shape=jax.ShapeDtypeStruct((1, 1), jnp.float32),
        out_specs=pl.BlockSpec(memory_space=pltpu.SMEM),
    )(part, target)


def kernel(target, pre_offset):
    pre_flat = pre_offset.reshape(-1)
    part = _make_sc_partials()(pre_flat)
    loss = _tc_combine(part, target)
    return loss[0, 0]


# speedup vs baseline: 1.9684x; 1.9684x over previous
import functools

import jax
import jax.numpy as jnp
from jax import lax
from jax.experimental import pallas as pl
from jax.experimental.pallas import tpu as pltpu
from jax.experimental.pallas import tpu_sc as plsc

H = 512
W = 512
PLANE = H * W
ROWS_BUF = 66
BUF = ROWS_BUF * W
PAD = 16
BUFA = BUF + 2 * PAD
NLEV = 3
NBATCH = 4
NSTRIP = 8
NW = 32


def _sc_partials_body(pre_hbm, part_hbm, buf0, buf1, obuf, sem0, sem1):
    cid = lax.axis_index("c")
    sid = lax.axis_index("s")
    wid = sid * 2 + cid
    n = wid // NSTRIP
    strip = wid % NSTRIP
    load_r0 = jnp.minimum(strip * 64, H - ROWS_BUF)
    br0 = jnp.where(strip == NSTRIP - 1, 3, 1)

    bufs = (buf0, buf1)
    sems = (sem0, sem1)

    def start(level):
        plane = ((level * NBATCH + n) * 3 + 2) * PLANE
        base = pl.multiple_of(plane + load_r0 * W, 512)
        return pltpu.async_copy(
            pre_hbm.at[pl.ds(base, BUF)],
            bufs[level % 2].at[pl.ds(PAD, BUF)],
            sems[level % 2],
        )

    iota = lax.iota(jnp.int32, 16)
    lanef = iota.astype(jnp.float32)

    cp = start(0)
    zero = jnp.zeros((16,), jnp.float32)
    base_rowf = load_r0.astype(jnp.float32)
    br0f = br0.astype(jnp.float32)
    SIv = zero
    SJv = zero
    CNTv = zero
    for level in range(NLEV):
        cp.wait()
        if level + 1 < NLEV:
            cp_next = start(level + 1)
        buf = bufs[level % 2]

        a_cnt = zero
        a_i = zero
        a_jb = zero
        for k in range(32):
            co = PAD + k * 16

            def ld(br, d, buf=buf, co=co):
                return buf[pl.ds(co + br * W + d, 16)]

            init = (ld(0, -1), ld(0, 0), ld(0, 1),
                    ld(1, -1), ld(1, 0), ld(1, 1),
                    a_cnt, a_i, a_jb)

            @plsc.parallel_loop(1, 65, unroll=2, carry=init)
            def _body(br, carry, buf=buf, co=co, k=k):
                ul, uc, ur, cl, cc, cr, b_cnt, b_i, b_jb = carry
                base = co + br * W
                nl = buf[pl.ds(base + W - 1, 16)]
                nc = buf[pl.ds(base + W, 16)]
                nr = buf[pl.ds(base + W + 1, 16)]
                mx = jnp.maximum(
                    jnp.maximum(jnp.maximum(ul, uc), jnp.maximum(ur, cl)),
                    jnp.maximum(jnp.maximum(cr, nl), jnp.maximum(nc, nr)),
                )
                m = cc > mx
                brf = br.astype(jnp.float32)
                if k == 0:
                    m = m & (iota > 0)
                elif k == 31:
                    m = m & (iota < 15)
                valid = jnp.where(brf >= br0f, jnp.float32(1.0), jnp.float32(0.0))
                mf = jnp.where(m, valid, jnp.float32(0.0))
                b_cnt = b_cnt + mf
                b_i = b_i + mf * (base_rowf + brf)
                if k > 0:
                    b_jb = b_jb + mf * jnp.float32(16.0 * k)
                return (cl, cc, cr, nl, nc, nr, b_cnt, b_i, b_jb)

            a_cnt, a_i, a_jb = _body[6], _body[7], _body[8]

        R = jnp.float32(4.0 * (2 ** level))
        SIv = SIv + R * a_i
        SJv = SJv + R * (a_jb + lanef * a_cnt)
        CNTv = CNTv + a_cnt
        if level + 1 < NLEV:
            cp = cp_next

    obuf[pl.ds(0, 16)] = SIv
    obuf[pl.ds(16, 16)] = SJv
    obuf[pl.ds(32, 16)] = CNTv
    pltpu.sync_copy(obuf, part_hbm.at[wid])


def _make_sc_partials():
    mesh = plsc.VectorSubcoreMesh(
        core_axis_name="c", subcore_axis_name="s", num_cores=2, num_subcores=16
    )
    return pl.kernel(
        _sc_partials_body,
        out_type=jax.ShapeDtypeStruct((NW, 48), jnp.float32),
        mesh=mesh,
        scratch_types=[
            pltpu.VMEM((BUFA,), jnp.float32),
            pltpu.VMEM((BUFA,), jnp.float32),
            pltpu.VMEM((48,), jnp.float32),
            pltpu.SemaphoreType.DMA,
            pltpu.SemaphoreType.DMA,
        ],
    )


def _tc_combine_body(part_ref, t_ref, out_ref):
    p = part_ref[...]
    t = t_ref[...]
    seg = lax.broadcasted_iota(jnp.int32, (NW, 48), 1) // 16
    grp = lax.broadcasted_iota(jnp.int32, (NW, 48), 0) // NSTRIP
    nrow = lax.broadcasted_iota(jnp.int32, (NBATCH, 200), 0)
    cx = (t[:, :, 0] + t[:, :, 2]) * 0.5
    cy = (t[:, :, 1] + t[:, :, 3]) * 0.5

    zero = jnp.float32(0.0)
    off_x = zero
    off_y = zero
    cs_tx = zero
    cs_ty = zero
    ts_tx = zero
    ts_ty = zero
    point_sum = zero
    for nn in range(NBATCH):
        mrow = grp == nn
        si_n = jnp.sum(jnp.where(mrow & (seg == 0), p, 0.0))
        sj_n = jnp.sum(jnp.where(mrow & (seg == 1), p, 0.0))
        c_n = jnp.sum(jnp.where(mrow & (seg == 2), p, 0.0))
        tx_n = jnp.sum(jnp.where(nrow == nn, cx, 0.0))
        ty_n = jnp.sum(jnp.where(nrow == nn, cy, 0.0))
        dx = jnp.abs(si_n - tx_n)
        dy = jnp.abs(sj_n - ty_n)
        off_x = off_x + jnp.where(dx < 1.0, 0.5 * dx * dx, dx - 0.5)
        off_y = off_y + jnp.where(dy < 1.0, 0.5 * dy * dy, dy - 0.5)
        cs_tx = cs_tx + si_n
        cs_ty = cs_ty + sj_n
        ts_tx = ts_tx + tx_n
        ts_ty = ts_ty + ty_n
        point_sum = point_sum + c_n
    loss = (off_x / jnp.abs(off_x) * (cs_tx - ts_tx)
            + off_y / jnp.abs(off_y) * (cs_ty - ts_ty)) / point_sum
    out_ref[0, 0] = loss


def _tc_combine(part, target):
    return pl.pallas_call(
        _tc_combine_body,
        out_shape=jax.ShapeDtypeStruct((1, 1), jnp.float32),
        out_specs=pl.BlockSpec(memory_space=pltpu.SMEM),
    )(part, target)


def kernel(target, pre_offset):
    pre_flat = pre_offset.reshape(-1)
    part = _make_sc_partials()(pre_flat)
    loss = _tc_combine(part, target)
    return loss[0, 0]
